# baseline (speedup 1.0000x reference)
import functools

import jax
import jax.numpy as jnp
from jax import lax
from jax.experimental import pallas as pl
from jax.experimental.pallas import tpu as pltpu
from jax.experimental.pallas import tpu_sc as plsc

NN = 10000
HALF = 5000
PADW = 120
NP = 10240
EE = 320000
FEAT = 128
SUB = 128
NU = 1280
NTILES = 16
UPT = NU // NTILES
ROWS_PT = NP // NTILES
ZB = 64
BR = 512
GRID = NP // BR

_mesh = plsc.VectorSubcoreMesh(core_axis_name="c", subcore_axis_name="s")


HUPT = UPT // 2


def _scatter_body(y_ref, srcp_ref, dstp_ref, zeros_ref, s_out,
                  acc, idxs_h, idxd_h, rows, semg, sem_idx):
    c = lax.axis_index("c")
    s = lax.axis_index("s")
    cz = pltpu.async_copy(zeros_ref, acc.at[pl.ds(s * ROWS_PT, ROWS_PT)],
                          sem_idx)
    pltpu.sync_copy(srcp_ref.at[c, pl.ds(s * UPT, HUPT)], idxs_h)
    pltpu.sync_copy(dstp_ref.at[c, pl.ds(s * UPT, HUPT)], idxd_h)

    def gather_start(lj, b):
        return pltpu.async_copy(y_ref.at[idxs_h.at[lj]], rows.at[b], semg)

    def gather_wait(lj, b):
        pltpu.make_async_copy(y_ref.at[idxs_h.at[lj]], rows.at[b],
                              semg).wait()

    def scat(lj, b):
        pltpu.sync_copy(rows.at[b], acc.at[idxd_h.at[lj]], add=True)

    last = HUPT - 1
    gather_start(0, 0)
    gather_start(1, 1)
    cz.wait()
    plsc.subcore_barrier()
    for h in range(2):
        if h == 1:
            pltpu.sync_copy(srcp_ref.at[c, pl.ds(s * UPT + HUPT, HUPT)],
                            idxs_h)
            pltpu.sync_copy(dstp_ref.at[c, pl.ds(s * UPT + HUPT, HUPT)],
                            idxd_h)
            gather_start(0, 0)
            gather_start(1, 1)

        def body(m, carry):
            j0 = 2 * m
            gather_wait(j0, 0)
            scat(j0, 0)
            gather_start(jnp.minimum(j0 + 2, last), 0)
            gather_wait(j0 + 1, 1)
            scat(j0 + 1, 1)
            gather_start(jnp.minimum(j0 + 3, last), 1)
            return carry

        lax.fori_loop(0, HUPT // 2, body, 0)
        gather_wait(last, 0)
        gather_wait(last, 1)
    plsc.subcore_barrier()
    pltpu.sync_copy(acc.at[pl.ds(s * ROWS_PT, ROWS_PT)],
                    s_out.at[c, pl.ds(s * ROWS_PT, ROWS_PT)])


_scatter_call = pl.kernel(
    _scatter_body,
    out_type=jax.ShapeDtypeStruct((2, NP, FEAT), jnp.float32),
    mesh=_mesh,
    scratch_types=[
        pltpu.VMEM_SHARED((NP, FEAT), jnp.float32),
        pltpu.VMEM((HUPT, SUB), jnp.int32),
        pltpu.VMEM((HUPT, SUB), jnp.int32),
        pltpu.VMEM((2, SUB, FEAT), jnp.float32),
        pltpu.SemaphoreType.DMA,
        pltpu.SemaphoreType.DMA,
    ],
)


def _row_spec(shape):
    return pl.BlockSpec(shape, lambda i: (i,) + (0,) * (len(shape) - 1))


def _full_spec(shape):
    return pl.BlockSpec(shape, lambda i: (0,) * len(shape))


def _prep_body(dega_ref, degb_ref, x_ref, w_ref, d1_ref, y0_ref, rst_ref):
    deg = dega_ref[:, 0:1] + degb_ref[:, 0:1]
    d1 = lax.rsqrt(jnp.maximum(deg, 1.0))
    d1_ref[...] = jnp.broadcast_to(d1, (BR, FEAT))
    x = x_ref[...]
    y0_ref[...] = x * d1
    rst_ref[...] = jnp.dot(x, w_ref[...], preferred_element_type=jnp.float32)


_prep_call = pl.pallas_call(
    _prep_body,
    grid=(GRID,),
    in_specs=[
        _row_spec((BR, FEAT)),
        _row_spec((BR, FEAT)),
        _row_spec((BR, FEAT)),
        _full_spec((FEAT, FEAT)),
    ],
    out_specs=[
        _row_spec((BR, FEAT)),
        _row_spec((BR, FEAT)),
        _row_spec((BR, FEAT)),
    ],
    out_shape=[jax.ShapeDtypeStruct((NP, FEAT), jnp.float32)] * 3,
)


def _make_step(a, with_xp, emit_x):
    def body(*refs):
        if with_xp:
            s0_ref, s1_ref, d1_ref, xp_ref, rst_ref, w_ref = refs[:6]
            outs = refs[6:]
        else:
            s0_ref, s1_ref, d1_ref, rst_ref, w_ref = refs[:5]
            outs = refs[5:]
        d1 = d1_ref[...]
        x = a * (d1 * (s0_ref[...] + s1_ref[...]))
        if with_xp:
            x = x - xp_ref[...]
        if emit_x:
            x_out, y_out, rst_out = outs
            x_out[...] = x
        else:
            y_out, rst_out = outs
        y_out[...] = d1 * x
        rst_out[...] = rst_ref[...] + jnp.dot(
            x, w_ref[...], preferred_element_type=jnp.float32)

    n_in = 6 if with_xp else 5
    n_out = 3 if emit_x else 2
    in_specs = [_row_spec((BR, FEAT))] * (n_in - 1) + [_full_spec((FEAT, FEAT))]
    return pl.pallas_call(
        body,
        grid=(GRID,),
        in_specs=in_specs,
        out_specs=[_row_spec((BR, FEAT))] * n_out,
        out_shape=[jax.ShapeDtypeStruct((NP, FEAT), jnp.float32)] * n_out,
    )


_step1_call = _make_step(-1.0, with_xp=False, emit_x=True)
_step2_call = _make_step(-2.0, with_xp=True, emit_x=True)
_step3_call = _make_step(-2.0, with_xp=True, emit_x=False)


def _layer_end_body(s0_ref, s1_ref, d1_ref, xp_ref, rst_ref, w4_ref, b_ref,
                    w0n_ref, h_ref, y0_ref, rst0_ref):
    d1 = d1_ref[...]
    x4 = -2.0 * (d1 * (s0_ref[...] + s1_ref[...])) - xp_ref[...]
    h = rst_ref[...] + jnp.dot(x4, w4_ref[...],
                               preferred_element_type=jnp.float32)
    h = jnp.maximum(h + b_ref[...], 0.0)
    h_ref[...] = h
    y0_ref[...] = d1 * h
    rst0_ref[...] = jnp.dot(h, w0n_ref[...],
                            preferred_element_type=jnp.float32)


_layer_end_call = pl.pallas_call(
    _layer_end_body,
    grid=(GRID,),
    in_specs=[
        _row_spec((BR, FEAT)),
        _row_spec((BR, FEAT)),
        _row_spec((BR, FEAT)),
        _row_spec((BR, FEAT)),
        _row_spec((BR, FEAT)),
        _full_spec((FEAT, FEAT)),
        _full_spec((1, FEAT)),
        _full_spec((FEAT, FEAT)),
    ],
    out_specs=[_row_spec((BR, FEAT))] * 3,
    out_shape=[jax.ShapeDtypeStruct((NP, FEAT), jnp.float32)] * 3,
)


def _final_end_body(s0_ref, s1_ref, d1_ref, xp_ref, rst_ref, w4_ref, b_ref,
                    wl_ref, bl_ref, out_ref):
    d1 = d1_ref[...]
    x4 = -2.0 * (d1 * (s0_ref[...] + s1_ref[...])) - xp_ref[...]
    h = rst_ref[...] + jnp.dot(x4, w4_ref[...],
                               preferred_element_type=jnp.float32)
    h = jnp.maximum(h + b_ref[...], 0.0)
    out_ref[...] = jnp.dot(h, wl_ref[...],
                           preferred_element_type=jnp.float32) + bl_ref[...]


_final_end_call = pl.pallas_call(
    _final_end_body,
    grid=(GRID,),
    in_specs=[
        _row_spec((BR, FEAT)),
        _row_spec((BR, FEAT)),
        _row_spec((BR, FEAT)),
        _row_spec((BR, FEAT)),
        _row_spec((BR, FEAT)),
        _full_spec((FEAT, FEAT)),
        _full_spec((1, FEAT)),
        _full_spec((FEAT, FEAT)),
        _full_spec((1, FEAT)),
    ],
    out_specs=[_row_spec((BR, FEAT))],
    out_shape=[jax.ShapeDtypeStruct((NP, FEAT), jnp.float32)],
)


def kernel(in_feat, edge_index, W1, b1, W2, b2, W3, b3, W4, b4, W5, b5, Wl, bl):
    f32 = jnp.float32
    i32 = jnp.int32

    src = edge_index[0].astype(i32)
    dst = edge_index[1].astype(i32)
    srcp = jnp.where(src >= HALF, src + PADW, src)
    dstp = jnp.where(dst >= HALF, dst + PADW, dst)
    npad = NU * SUB - EE // 2
    src_arr = jnp.concatenate(
        [srcp.reshape(2, EE // 2),
         jnp.zeros((2, npad), i32)], axis=1).reshape(2, NU, SUB)
    dst_arr = jnp.concatenate(
        [dstp.reshape(2, EE // 2),
         jnp.full((2, npad), NP - 1, i32)], axis=1).reshape(2, NU, SUB)

    zeros128 = jnp.zeros((ROWS_PT, FEAT), f32)

    pad_rows = jnp.zeros((PADW, FEAT), f32)
    x0 = jnp.concatenate(
        [in_feat[:HALF], pad_rows, in_feat[HALF:], pad_rows], axis=0)

    ones_feat = jnp.ones((NP, FEAT), f32)
    deg = _scatter_call(ones_feat, src_arr, dst_arr, zeros128)
    d1e, y, rst = _prep_call(deg[0], deg[1], x0, W1[0])

    layers = [(W1, b1), (W2, b2), (W3, b3), (W4, b4), (W5, b5)]
    for li, (W, b) in enumerate(layers):
        b2d = b.reshape(1, FEAT)
        s = _scatter_call(y, src_arr, dst_arr, zeros128)
        x1, y, rst = _step1_call(s[0], s[1], d1e, rst, W[1])
        s = _scatter_call(y, src_arr, dst_arr, zeros128)
        x2, y, rst = _step2_call(s[0], s[1], d1e, x0, rst, W[2])
        s = _scatter_call(y, src_arr, dst_arr, zeros128)
        y, rst = _step3_call(s[0], s[1], d1e, x1, rst, W[3])
        s = _scatter_call(y, src_arr, dst_arr, zeros128)
        if li < 4:
            w0n = layers[li + 1][0][0]
            x0, y, rst = _layer_end_call(
                s[0], s[1], d1e, x2, rst, W[4], b2d, w0n)
        else:
            wl_pad = jnp.zeros((FEAT, FEAT), f32).at[:, :Wl.shape[1]].set(Wl)
            bl_pad = jnp.zeros((1, FEAT), f32).at[0, :bl.shape[0]].set(bl)
            out_p, = _final_end_call(
                s[0], s[1], d1e, x2, rst, W[4], b2d, wl_pad, bl_pad)

    return jnp.concatenate(
        [out_p[:HALF, :Wl.shape[1]],
         out_p[HALF + PADW:HALF + PADW + HALF, :Wl.shape[1]]], axis=0)


# speedup vs baseline: 1.0278x; 1.0278x over previous
import functools

import jax
import jax.numpy as jnp
from jax import lax
from jax.experimental import pallas as pl
from jax.experimental.pallas import tpu as pltpu
from jax.experimental.pallas import tpu_sc as plsc

NN = 10000
HALF = 5000
PADW = 120
NP = 10240
EE = 320000
FEAT = 128
SUB = 128
NU = 1280
NTILES = 16
UPT = NU // NTILES
ROWS_PT = NP // NTILES
ZB = 64
BR = 512
GRID = NP // BR

_mesh = plsc.VectorSubcoreMesh(core_axis_name="c", subcore_axis_name="s")


HUPT = UPT // 2


def _scatter_body(y_ref, srcp_ref, dstp_ref, zeros_ref, s_out,
                  acc, idxs_h, idxd_h, rows, semg, sem_idx):
    c = lax.axis_index("c")
    s = lax.axis_index("s")
    cz = pltpu.async_copy(zeros_ref, acc.at[pl.ds(s * ROWS_PT, ROWS_PT)],
                          sem_idx)
    pltpu.sync_copy(srcp_ref.at[c, pl.ds(s * UPT, HUPT)], idxs_h)
    pltpu.sync_copy(dstp_ref.at[c, pl.ds(s * UPT, HUPT)], idxd_h)

    def gather_start(lj, b):
        return pltpu.async_copy(y_ref.at[idxs_h.at[lj]], rows.at[b], semg)

    def gather_wait(lj, b):
        pltpu.make_async_copy(y_ref.at[idxs_h.at[lj]], rows.at[b],
                              semg).wait()

    def scat(lj, b):
        pltpu.sync_copy(rows.at[b], acc.at[idxd_h.at[lj]], add=True)

    last = HUPT - 1
    gather_start(0, 0)
    cz.wait()
    plsc.subcore_barrier()
    for h in range(2):
        if h == 1:
            pltpu.sync_copy(srcp_ref.at[c, pl.ds(s * UPT + HUPT, HUPT)],
                            idxs_h)
            pltpu.sync_copy(dstp_ref.at[c, pl.ds(s * UPT + HUPT, HUPT)],
                            idxd_h)
            gather_start(0, 0)

        def body(m, carry):
            j0 = 2 * m
            gather_start(j0 + 1, 1)
            gather_wait(j0, 0)
            scat(j0, 0)
            gather_start(jnp.minimum(j0 + 2, last), 0)
            gather_wait(j0 + 1, 1)
            scat(j0 + 1, 1)
            return carry

        lax.fori_loop(0, HUPT // 2, body, 0)
        gather_wait(last, 0)
    plsc.subcore_barrier()
    pltpu.sync_copy(acc.at[pl.ds(s * ROWS_PT, ROWS_PT)],
                    s_out.at[c, pl.ds(s * ROWS_PT, ROWS_PT)])


_scatter_call = pl.kernel(
    _scatter_body,
    out_type=jax.ShapeDtypeStruct((2, NP, FEAT), jnp.float32),
    mesh=_mesh,
    scratch_types=[
        pltpu.VMEM_SHARED((NP, FEAT), jnp.float32),
        pltpu.VMEM((HUPT, SUB), jnp.int32),
        pltpu.VMEM((HUPT, SUB), jnp.int32),
        pltpu.VMEM((2, SUB, FEAT), jnp.float32),
        pltpu.SemaphoreType.DMA,
        pltpu.SemaphoreType.DMA,
    ],
)


def _row_spec(shape):
    return pl.BlockSpec(shape, lambda i: (i,) + (0,) * (len(shape) - 1))


def _full_spec(shape):
    return pl.BlockSpec(shape, lambda i: (0,) * len(shape))


def _prep_body(dega_ref, degb_ref, x_ref, w_ref, d1_ref, y0_ref, rst_ref):
    deg = dega_ref[:, 0:1] + degb_ref[:, 0:1]
    d1 = lax.rsqrt(jnp.maximum(deg, 1.0))
    d1_ref[...] = jnp.broadcast_to(d1, (BR, FEAT))
    x = x_ref[...]
    y0_ref[...] = x * d1
    rst_ref[...] = jnp.dot(x, w_ref[...], preferred_element_type=jnp.float32)


_prep_call = pl.pallas_call(
    _prep_body,
    grid=(GRID,),
    in_specs=[
        _row_spec((BR, FEAT)),
        _row_spec((BR, FEAT)),
        _row_spec((BR, FEAT)),
        _full_spec((FEAT, FEAT)),
    ],
    out_specs=[
        _row_spec((BR, FEAT)),
        _row_spec((BR, FEAT)),
        _row_spec((BR, FEAT)),
    ],
    out_shape=[jax.ShapeDtypeStruct((NP, FEAT), jnp.float32)] * 3,
)


def _make_step(a, with_xp, emit_x):
    def body(*refs):
        if with_xp:
            s0_ref, s1_ref, d1_ref, xp_ref, rst_ref, w_ref = refs[:6]
            outs = refs[6:]
        else:
            s0_ref, s1_ref, d1_ref, rst_ref, w_ref = refs[:5]
            outs = refs[5:]
        d1 = d1_ref[...]
        x = a * (d1 * (s0_ref[...] + s1_ref[...]))
        if with_xp:
            x = x - xp_ref[...]
        if emit_x:
            x_out, y_out, rst_out = outs
            x_out[...] = x
        else:
            y_out, rst_out = outs
        y_out[...] = d1 * x
        rst_out[...] = rst_ref[...] + jnp.dot(
            x, w_ref[...], preferred_element_type=jnp.float32)

    n_in = 6 if with_xp else 5
    n_out = 3 if emit_x else 2
    in_specs = [_row_spec((BR, FEAT))] * (n_in - 1) + [_full_spec((FEAT, FEAT))]
    return pl.pallas_call(
        body,
        grid=(GRID,),
        in_specs=in_specs,
        out_specs=[_row_spec((BR, FEAT))] * n_out,
        out_shape=[jax.ShapeDtypeStruct((NP, FEAT), jnp.float32)] * n_out,
    )


_step1_call = _make_step(-1.0, with_xp=False, emit_x=True)
_step2_call = _make_step(-2.0, with_xp=True, emit_x=True)
_step3_call = _make_step(-2.0, with_xp=True, emit_x=False)


def _layer_end_body(s0_ref, s1_ref, d1_ref, xp_ref, rst_ref, w4_ref, b_ref,
                    w0n_ref, h_ref, y0_ref, rst0_ref):
    d1 = d1_ref[...]
    x4 = -2.0 * (d1 * (s0_ref[...] + s1_ref[...])) - xp_ref[...]
    h = rst_ref[...] + jnp.dot(x4, w4_ref[...],
                               preferred_element_type=jnp.float32)
    h = jnp.maximum(h + b_ref[...], 0.0)
    h_ref[...] = h
    y0_ref[...] = d1 * h
    rst0_ref[...] = jnp.dot(h, w0n_ref[...],
                            preferred_element_type=jnp.float32)


_layer_end_call = pl.pallas_call(
    _layer_end_body,
    grid=(GRID,),
    in_specs=[
        _row_spec((BR, FEAT)),
        _row_spec((BR, FEAT)),
        _row_spec((BR, FEAT)),
        _row_spec((BR, FEAT)),
        _row_spec((BR, FEAT)),
        _full_spec((FEAT, FEAT)),
        _full_spec((1, FEAT)),
        _full_spec((FEAT, FEAT)),
    ],
    out_specs=[_row_spec((BR, FEAT))] * 3,
    out_shape=[jax.ShapeDtypeStruct((NP, FEAT), jnp.float32)] * 3,
)


def _final_end_body(s0_ref, s1_ref, d1_ref, xp_ref, rst_ref, w4_ref, b_ref,
                    wl_ref, bl_ref, out_ref):
    d1 = d1_ref[...]
    x4 = -2.0 * (d1 * (s0_ref[...] + s1_ref[...])) - xp_ref[...]
    h = rst_ref[...] + jnp.dot(x4, w4_ref[...],
                               preferred_element_type=jnp.float32)
    h = jnp.maximum(h + b_ref[...], 0.0)
    out_ref[...] = jnp.dot(h, wl_ref[...],
                           preferred_element_type=jnp.float32) + bl_ref[...]


_final_end_call = pl.pallas_call(
    _final_end_body,
    grid=(GRID,),
    in_specs=[
        _row_spec((BR, FEAT)),
        _row_spec((BR, FEAT)),
        _row_spec((BR, FEAT)),
        _row_spec((BR, FEAT)),
        _row_spec((BR, FEAT)),
        _full_spec((FEAT, FEAT)),
        _full_spec((1, FEAT)),
        _full_spec((FEAT, FEAT)),
        _full_spec((1, FEAT)),
    ],
    out_specs=[_row_spec((BR, FEAT))],
    out_shape=[jax.ShapeDtypeStruct((NP, FEAT), jnp.float32)],
)


def kernel(in_feat, edge_index, W1, b1, W2, b2, W3, b3, W4, b4, W5, b5, Wl, bl):
    f32 = jnp.float32
    i32 = jnp.int32

    src = edge_index[0].astype(i32)
    dst = edge_index[1].astype(i32)
    srcp = jnp.where(src >= HALF, src + PADW, src)
    dstp = jnp.where(dst >= HALF, dst + PADW, dst)
    npad = NU * SUB - EE // 2
    src_arr = jnp.concatenate(
        [srcp.reshape(2, EE // 2),
         jnp.zeros((2, npad), i32)], axis=1).reshape(2, NU, SUB)
    dst_arr = jnp.concatenate(
        [dstp.reshape(2, EE // 2),
         jnp.full((2, npad), NP - 1, i32)], axis=1).reshape(2, NU, SUB)

    zeros128 = jnp.zeros((ROWS_PT, FEAT), f32)

    pad_rows = jnp.zeros((PADW, FEAT), f32)
    x0 = jnp.concatenate(
        [in_feat[:HALF], pad_rows, in_feat[HALF:], pad_rows], axis=0)

    ones_feat = jnp.ones((NP, FEAT), f32)
    deg = _scatter_call(ones_feat, src_arr, dst_arr, zeros128)
    d1e, y, rst = _prep_call(deg[0], deg[1], x0, W1[0])

    layers = [(W1, b1), (W2, b2), (W3, b3), (W4, b4), (W5, b5)]
    for li, (W, b) in enumerate(layers):
        b2d = b.reshape(1, FEAT)
        s = _scatter_call(y, src_arr, dst_arr, zeros128)
        x1, y, rst = _step1_call(s[0], s[1], d1e, rst, W[1])
        s = _scatter_call(y, src_arr, dst_arr, zeros128)
        x2, y, rst = _step2_call(s[0], s[1], d1e, x0, rst, W[2])
        s = _scatter_call(y, src_arr, dst_arr, zeros128)
        y, rst = _step3_call(s[0], s[1], d1e, x1, rst, W[3])
        s = _scatter_call(y, src_arr, dst_arr, zeros128)
        if li < 4:
            w0n = layers[li + 1][0][0]
            x0, y, rst = _layer_end_call(
                s[0], s[1], d1e, x2, rst, W[4], b2d, w0n)
        else:
            wl_pad = jnp.zeros((FEAT, FEAT), f32).at[:, :Wl.shape[1]].set(Wl)
            bl_pad = jnp.zeros((1, FEAT), f32).at[0, :bl.shape[0]].set(bl)
            out_p, = _final_end_call(
                s[0], s[1], d1e, x2, rst, W[4], b2d, wl_pad, bl_pad)

    return jnp.concatenate(
        [out_p[:HALF, :Wl.shape[1]],
         out_p[HALF + PADW:HALF + PADW + HALF, :Wl.shape[1]]], axis=0)


# speedup vs baseline: 1.0407x; 1.0125x over previous
import functools

import jax
import jax.numpy as jnp
from jax import lax
from jax.experimental import pallas as pl
from jax.experimental.pallas import tpu as pltpu
from jax.experimental.pallas import tpu_sc as plsc

NN = 10000
HALF = 5000
PADW = 120
NP = 10240
EE = 320000
FEAT = 128
SUB = 128
NU = 1280
NTILES = 16
UPT = NU // NTILES
ROWS_PT = NP // NTILES
ZB = 64
BR = 1024
GRID = NP // BR

_mesh = plsc.VectorSubcoreMesh(core_axis_name="c", subcore_axis_name="s")


HUPT = UPT // 2


def _scatter_body(y_ref, srcp_ref, dstp_ref, zeros_ref, s_out,
                  acc, idxs_h, idxd_h, rows, semg, sem_idx):
    c = lax.axis_index("c")
    s = lax.axis_index("s")
    cz = pltpu.async_copy(zeros_ref, acc.at[pl.ds(s * ROWS_PT, ROWS_PT)],
                          sem_idx)
    pltpu.sync_copy(srcp_ref.at[c, pl.ds(s * UPT, HUPT)], idxs_h)
    pltpu.sync_copy(dstp_ref.at[c, pl.ds(s * UPT, HUPT)], idxd_h)

    def gather_start(lj, b):
        return pltpu.async_copy(y_ref.at[idxs_h.at[lj]], rows.at[b], semg)

    def gather_wait(lj, b):
        pltpu.make_async_copy(y_ref.at[idxs_h.at[lj]], rows.at[b],
                              semg).wait()

    def scat(lj, b):
        pltpu.sync_copy(rows.at[b], acc.at[idxd_h.at[lj]], add=True)

    last = HUPT - 1
    gather_start(0, 0)
    cz.wait()
    plsc.subcore_barrier()
    for h in range(2):
        if h == 1:
            pltpu.sync_copy(srcp_ref.at[c, pl.ds(s * UPT + HUPT, HUPT)],
                            idxs_h)
            pltpu.sync_copy(dstp_ref.at[c, pl.ds(s * UPT + HUPT, HUPT)],
                            idxd_h)
            gather_start(0, 0)

        def body(m, carry):
            j0 = 2 * m
            gather_start(j0 + 1, 1)
            gather_wait(j0, 0)
            scat(j0, 0)
            gather_start(jnp.minimum(j0 + 2, last), 0)
            gather_wait(j0 + 1, 1)
            scat(j0 + 1, 1)
            return carry

        lax.fori_loop(0, HUPT // 2, body, 0)
        gather_wait(last, 0)
    plsc.subcore_barrier()
    pltpu.sync_copy(acc.at[pl.ds(s * ROWS_PT, ROWS_PT)],
                    s_out.at[c, pl.ds(s * ROWS_PT, ROWS_PT)])


_scatter_call = pl.kernel(
    _scatter_body,
    out_type=jax.ShapeDtypeStruct((2, NP, FEAT), jnp.float32),
    mesh=_mesh,
    scratch_types=[
        pltpu.VMEM_SHARED((NP, FEAT), jnp.float32),
        pltpu.VMEM((HUPT, SUB), jnp.int32),
        pltpu.VMEM((HUPT, SUB), jnp.int32),
        pltpu.VMEM((2, SUB, FEAT), jnp.float32),
        pltpu.SemaphoreType.DMA,
        pltpu.SemaphoreType.DMA,
    ],
)


def _row_spec(shape):
    return pl.BlockSpec(shape, lambda i: (i,) + (0,) * (len(shape) - 1))


def _full_spec(shape):
    return pl.BlockSpec(shape, lambda i: (0,) * len(shape))


def _prep_body(dega_ref, degb_ref, x_ref, w_ref, d1_ref, y0_ref, rst_ref):
    deg = dega_ref[:, 0:1] + degb_ref[:, 0:1]
    d1 = lax.rsqrt(jnp.maximum(deg, 1.0))
    d1_ref[...] = jnp.broadcast_to(d1, (BR, FEAT))
    x = x_ref[...]
    y0_ref[...] = x * d1
    rst_ref[...] = jnp.dot(x, w_ref[...], preferred_element_type=jnp.float32)


_prep_call = pl.pallas_call(
    _prep_body,
    grid=(GRID,),
    in_specs=[
        _row_spec((BR, FEAT)),
        _row_spec((BR, FEAT)),
        _row_spec((BR, FEAT)),
        _full_spec((FEAT, FEAT)),
    ],
    out_specs=[
        _row_spec((BR, FEAT)),
        _row_spec((BR, FEAT)),
        _row_spec((BR, FEAT)),
    ],
    out_shape=[jax.ShapeDtypeStruct((NP, FEAT), jnp.float32)] * 3,
)


def _make_step(a, with_xp, emit_x):
    def body(*refs):
        if with_xp:
            s0_ref, s1_ref, d1_ref, xp_ref, rst_ref, w_ref = refs[:6]
            outs = refs[6:]
        else:
            s0_ref, s1_ref, d1_ref, rst_ref, w_ref = refs[:5]
            outs = refs[5:]
        d1 = d1_ref[...]
        x = a * (d1 * (s0_ref[...] + s1_ref[...]))
        if with_xp:
            x = x - xp_ref[...]
        if emit_x:
            x_out, y_out, rst_out = outs
            x_out[...] = x
        else:
            y_out, rst_out = outs
        y_out[...] = d1 * x
        rst_out[...] = rst_ref[...] + jnp.dot(
            x, w_ref[...], preferred_element_type=jnp.float32)

    n_in = 6 if with_xp else 5
    n_out = 3 if emit_x else 2
    in_specs = [_row_spec((BR, FEAT))] * (n_in - 1) + [_full_spec((FEAT, FEAT))]
    return pl.pallas_call(
        body,
        grid=(GRID,),
        in_specs=in_specs,
        out_specs=[_row_spec((BR, FEAT))] * n_out,
        out_shape=[jax.ShapeDtypeStruct((NP, FEAT), jnp.float32)] * n_out,
    )


_step1_call = _make_step(-1.0, with_xp=False, emit_x=True)
_step2_call = _make_step(-2.0, with_xp=True, emit_x=True)
_step3_call = _make_step(-2.0, with_xp=True, emit_x=False)


def _layer_end_body(s0_ref, s1_ref, d1_ref, xp_ref, rst_ref, w4_ref, b_ref,
                    w0n_ref, h_ref, y0_ref, rst0_ref):
    d1 = d1_ref[...]
    x4 = -2.0 * (d1 * (s0_ref[...] + s1_ref[...])) - xp_ref[...]
    h = rst_ref[...] + jnp.dot(x4, w4_ref[...],
                               preferred_element_type=jnp.float32)
    h = jnp.maximum(h + b_ref[...], 0.0)
    h_ref[...] = h
    y0_ref[...] = d1 * h
    rst0_ref[...] = jnp.dot(h, w0n_ref[...],
                            preferred_element_type=jnp.float32)


_layer_end_call = pl.pallas_call(
    _layer_end_body,
    grid=(GRID,),
    in_specs=[
        _row_spec((BR, FEAT)),
        _row_spec((BR, FEAT)),
        _row_spec((BR, FEAT)),
        _row_spec((BR, FEAT)),
        _row_spec((BR, FEAT)),
        _full_spec((FEAT, FEAT)),
        _full_spec((1, FEAT)),
        _full_spec((FEAT, FEAT)),
    ],
    out_specs=[_row_spec((BR, FEAT))] * 3,
    out_shape=[jax.ShapeDtypeStruct((NP, FEAT), jnp.float32)] * 3,
)


def _final_end_body(s0_ref, s1_ref, d1_ref, xp_ref, rst_ref, w4_ref, b_ref,
                    wl_ref, bl_ref, out_ref):
    d1 = d1_ref[...]
    x4 = -2.0 * (d1 * (s0_ref[...] + s1_ref[...])) - xp_ref[...]
    h = rst_ref[...] + jnp.dot(x4, w4_ref[...],
                               preferred_element_type=jnp.float32)
    h = jnp.maximum(h + b_ref[...], 0.0)
    out_ref[...] = jnp.dot(h, wl_ref[...],
                           preferred_element_type=jnp.float32) + bl_ref[...]


_final_end_call = pl.pallas_call(
    _final_end_body,
    grid=(GRID,),
    in_specs=[
        _row_spec((BR, FEAT)),
        _row_spec((BR, FEAT)),
        _row_spec((BR, FEAT)),
        _row_spec((BR, FEAT)),
        _row_spec((BR, FEAT)),
        _full_spec((FEAT, FEAT)),
        _full_spec((1, FEAT)),
        _full_spec((FEAT, FEAT)),
        _full_spec((1, FEAT)),
    ],
    out_specs=[_row_spec((BR, FEAT))],
    out_shape=[jax.ShapeDtypeStruct((NP, FEAT), jnp.float32)],
)


def kernel(in_feat, edge_index, W1, b1, W2, b2, W3, b3, W4, b4, W5, b5, Wl, bl):
    f32 = jnp.float32
    i32 = jnp.int32

    src = edge_index[0].astype(i32)
    dst = edge_index[1].astype(i32)
    srcp = jnp.where(src >= HALF, src + PADW, src)
    dstp = jnp.where(dst >= HALF, dst + PADW, dst)
    npad = NU * SUB - EE // 2
    src_arr = jnp.concatenate(
        [srcp.reshape(2, EE // 2),
         jnp.zeros((2, npad), i32)], axis=1).reshape(2, NU, SUB)
    dst_arr = jnp.concatenate(
        [dstp.reshape(2, EE // 2),
         jnp.full((2, npad), NP - 1, i32)], axis=1).reshape(2, NU, SUB)

    zeros128 = jnp.zeros((ROWS_PT, FEAT), f32)

    pad_rows = jnp.zeros((PADW, FEAT), f32)
    x0 = jnp.concatenate(
        [in_feat[:HALF], pad_rows, in_feat[HALF:], pad_rows], axis=0)

    ones_feat = jnp.ones((NP, FEAT), f32)
    deg = _scatter_call(ones_feat, src_arr, dst_arr, zeros128)
    d1e, y, rst = _prep_call(deg[0], deg[1], x0, W1[0])

    layers = [(W1, b1), (W2, b2), (W3, b3), (W4, b4), (W5, b5)]
    for li, (W, b) in enumerate(layers):
        b2d = b.reshape(1, FEAT)
        s = _scatter_call(y, src_arr, dst_arr, zeros128)
        x1, y, rst = _step1_call(s[0], s[1], d1e, rst, W[1])
        s = _scatter_call(y, src_arr, dst_arr, zeros128)
        x2, y, rst = _step2_call(s[0], s[1], d1e, x0, rst, W[2])
        s = _scatter_call(y, src_arr, dst_arr, zeros128)
        y, rst = _step3_call(s[0], s[1], d1e, x1, rst, W[3])
        s = _scatter_call(y, src_arr, dst_arr, zeros128)
        if li < 4:
            w0n = layers[li + 1][0][0]
            x0, y, rst = _layer_end_call(
                s[0], s[1], d1e, x2, rst, W[4], b2d, w0n)
        else:
            wl_pad = jnp.zeros((FEAT, FEAT), f32).at[:, :Wl.shape[1]].set(Wl)
            bl_pad = jnp.zeros((1, FEAT), f32).at[0, :bl.shape[0]].set(bl)
            out_p, = _final_end_call(
                s[0], s[1], d1e, x2, rst, W[4], b2d, wl_pad, bl_pad)

    return jnp.concatenate(
        [out_p[:HALF, :Wl.shape[1]],
         out_p[HALF + PADW:HALF + PADW + HALF, :Wl.shape[1]]], axis=0)


# speedup vs baseline: 1.0422x; 1.0015x over previous
import functools

import jax
import jax.numpy as jnp
from jax import lax
from jax.experimental import pallas as pl
from jax.experimental.pallas import tpu as pltpu
from jax.experimental.pallas import tpu_sc as plsc

NN = 10000
HALF = 5000
PADW = 120
NP = 10240
EE = 320000
FEAT = 128
SUB = 128
NU = 1280
NTILES = 16
UPT = NU // NTILES
ROWS_PT = NP // NTILES
ZB = 64
BR = 1024
GRID = NP // BR

_mesh = plsc.VectorSubcoreMesh(core_axis_name="c", subcore_axis_name="s")


HUPT = UPT // 2


def _scatter_body(y_ref, srcp_ref, dstp_ref, zeros_ref, s_out,
                  acc, idxs_h, idxd_h, rows, semg, sem_idx):
    c = lax.axis_index("c")
    s = lax.axis_index("s")
    cz = pltpu.async_copy(zeros_ref, acc.at[pl.ds(s * ROWS_PT, ROWS_PT)],
                          sem_idx)
    pltpu.sync_copy(srcp_ref.at[c, pl.ds(s * UPT, HUPT)], idxs_h)
    pltpu.sync_copy(dstp_ref.at[c, pl.ds(s * UPT, HUPT)], idxd_h)

    def gather_start(lj, b):
        return pltpu.async_copy(y_ref.at[idxs_h.at[lj]], rows.at[b], semg)

    def gather_wait(lj, b):
        pltpu.make_async_copy(y_ref.at[idxs_h.at[lj]], rows.at[b],
                              semg).wait()

    def scat(lj, b):
        pltpu.sync_copy(rows.at[b], acc.at[idxd_h.at[lj]], add=True)

    last = HUPT - 1
    gather_start(0, 0)
    cz.wait()
    plsc.subcore_barrier()
    for h in range(2):
        if h == 1:
            pltpu.sync_copy(srcp_ref.at[c, pl.ds(s * UPT + HUPT, HUPT)],
                            idxs_h)
            pltpu.sync_copy(dstp_ref.at[c, pl.ds(s * UPT + HUPT, HUPT)],
                            idxd_h)
            gather_start(0, 0)

        def body(m, carry):
            j0 = 2 * m
            gather_start(j0 + 1, 1)
            gather_wait(j0, 0)
            scat(j0, 0)
            gather_start(jnp.minimum(j0 + 2, last), 0)
            gather_wait(j0 + 1, 1)
            scat(j0 + 1, 1)
            return carry

        lax.fori_loop(0, HUPT // 2, body, 0)
        gather_wait(last, 0)
    plsc.subcore_barrier()
    pltpu.sync_copy(acc.at[pl.ds(s * ROWS_PT, ROWS_PT)],
                    s_out.at[c, pl.ds(s * ROWS_PT, ROWS_PT)])


_scatter_call = pl.kernel(
    _scatter_body,
    out_type=jax.ShapeDtypeStruct((2, NP, FEAT), jnp.float32),
    mesh=_mesh,
    scratch_types=[
        pltpu.VMEM_SHARED((NP, FEAT), jnp.float32),
        pltpu.VMEM((HUPT, SUB), jnp.int32),
        pltpu.VMEM((HUPT, SUB), jnp.int32),
        pltpu.VMEM((2, SUB, FEAT), jnp.float32),
        pltpu.SemaphoreType.DMA,
        pltpu.SemaphoreType.DMA,
    ],
)


def _row_spec(shape):
    return pl.BlockSpec(shape, lambda i: (i,) + (0,) * (len(shape) - 1))


def _full_spec(shape):
    return pl.BlockSpec(shape, lambda i: (0,) * len(shape))


def _prep_body(dega_ref, degb_ref, x_ref, w_ref, d1_ref, y0_ref, rst_ref):
    deg = dega_ref[:, 0:1] + degb_ref[:, 0:1]
    d1 = lax.rsqrt(jnp.maximum(deg, 1.0))
    d1_ref[...] = jnp.broadcast_to(d1, (BR, FEAT))
    x = x_ref[...]
    y0_ref[...] = x * d1
    rst_ref[...] = jnp.dot(x, w_ref[...], preferred_element_type=jnp.float32)


_prep_call = pl.pallas_call(
    _prep_body,
    grid=(GRID,),
    in_specs=[
        _row_spec((BR, FEAT)),
        _row_spec((BR, FEAT)),
        _row_spec((BR, FEAT)),
        _full_spec((FEAT, FEAT)),
    ],
    out_specs=[
        _row_spec((BR, FEAT)),
        _row_spec((BR, FEAT)),
        _row_spec((BR, FEAT)),
    ],
    out_shape=[jax.ShapeDtypeStruct((NP, FEAT), jnp.float32)] * 3,
)


def _make_step_xy(a, with_xp):
    def body(*refs):
        if with_xp:
            s0_ref, s1_ref, d1_ref, xp_ref, x_out, y_out = refs
        else:
            s0_ref, s1_ref, d1_ref, x_out, y_out = refs
        d1 = d1_ref[...]
        x = a * (d1 * (s0_ref[...] + s1_ref[...]))
        if with_xp:
            x = x - xp_ref[...]
        x_out[...] = x
        y_out[...] = d1 * x

    n_in = 4 if with_xp else 3
    return pl.pallas_call(
        body,
        grid=(GRID,),
        in_specs=[_row_spec((BR, FEAT))] * n_in,
        out_specs=[_row_spec((BR, FEAT))] * 2,
        out_shape=[jax.ShapeDtypeStruct((NP, FEAT), jnp.float32)] * 2,
    )


_step1_call = _make_step_xy(-1.0, with_xp=False)
_step23_call = _make_step_xy(-2.0, with_xp=True)


def _mm_body(rst_ref, x_ref, w_ref, rst_out):
    rst_out[...] = rst_ref[...] + jnp.dot(
        x_ref[...], w_ref[...], preferred_element_type=jnp.float32)


_mm_call = pl.pallas_call(
    _mm_body,
    grid=(GRID,),
    in_specs=[
        _row_spec((BR, FEAT)),
        _row_spec((BR, FEAT)),
        _full_spec((FEAT, FEAT)),
    ],
    out_specs=[_row_spec((BR, FEAT))],
    out_shape=[jax.ShapeDtypeStruct((NP, FEAT), jnp.float32)],
)


def _layer_end_body(s0_ref, s1_ref, d1_ref, xp_ref, rst_ref, w4_ref, b_ref,
                    w0n_ref, h_ref, y0_ref, rst0_ref):
    d1 = d1_ref[...]
    x4 = -2.0 * (d1 * (s0_ref[...] + s1_ref[...])) - xp_ref[...]
    h = rst_ref[...] + jnp.dot(x4, w4_ref[...],
                               preferred_element_type=jnp.float32)
    h = jnp.maximum(h + b_ref[...], 0.0)
    h_ref[...] = h
    y0_ref[...] = d1 * h
    rst0_ref[...] = jnp.dot(h, w0n_ref[...],
                            preferred_element_type=jnp.float32)


_layer_end_call = pl.pallas_call(
    _layer_end_body,
    grid=(GRID,),
    in_specs=[
        _row_spec((BR, FEAT)),
        _row_spec((BR, FEAT)),
        _row_spec((BR, FEAT)),
        _row_spec((BR, FEAT)),
        _row_spec((BR, FEAT)),
        _full_spec((FEAT, FEAT)),
        _full_spec((1, FEAT)),
        _full_spec((FEAT, FEAT)),
    ],
    out_specs=[_row_spec((BR, FEAT))] * 3,
    out_shape=[jax.ShapeDtypeStruct((NP, FEAT), jnp.float32)] * 3,
)


def _final_end_body(s0_ref, s1_ref, d1_ref, xp_ref, rst_ref, w4_ref, b_ref,
                    wl_ref, bl_ref, out_ref):
    d1 = d1_ref[...]
    x4 = -2.0 * (d1 * (s0_ref[...] + s1_ref[...])) - xp_ref[...]
    h = rst_ref[...] + jnp.dot(x4, w4_ref[...],
                               preferred_element_type=jnp.float32)
    h = jnp.maximum(h + b_ref[...], 0.0)
    out_ref[...] = jnp.dot(h, wl_ref[...],
                           preferred_element_type=jnp.float32) + bl_ref[...]


_final_end_call = pl.pallas_call(
    _final_end_body,
    grid=(GRID,),
    in_specs=[
        _row_spec((BR, FEAT)),
        _row_spec((BR, FEAT)),
        _row_spec((BR, FEAT)),
        _row_spec((BR, FEAT)),
        _row_spec((BR, FEAT)),
        _full_spec((FEAT, FEAT)),
        _full_spec((1, FEAT)),
        _full_spec((FEAT, FEAT)),
        _full_spec((1, FEAT)),
    ],
    out_specs=[_row_spec((BR, FEAT))],
    out_shape=[jax.ShapeDtypeStruct((NP, FEAT), jnp.float32)],
)


def kernel(in_feat, edge_index, W1, b1, W2, b2, W3, b3, W4, b4, W5, b5, Wl, bl):
    f32 = jnp.float32
    i32 = jnp.int32

    src = edge_index[0].astype(i32)
    dst = edge_index[1].astype(i32)
    srcp = jnp.where(src >= HALF, src + PADW, src)
    dstp = jnp.where(dst >= HALF, dst + PADW, dst)
    npad = NU * SUB - EE // 2
    src_arr = jnp.concatenate(
        [srcp.reshape(2, EE // 2),
         jnp.zeros((2, npad), i32)], axis=1).reshape(2, NU, SUB)
    dst_arr = jnp.concatenate(
        [dstp.reshape(2, EE // 2),
         jnp.full((2, npad), NP - 1, i32)], axis=1).reshape(2, NU, SUB)

    zeros128 = jnp.zeros((ROWS_PT, FEAT), f32)

    pad_rows = jnp.zeros((PADW, FEAT), f32)
    x0 = jnp.concatenate(
        [in_feat[:HALF], pad_rows, in_feat[HALF:], pad_rows], axis=0)

    ones_feat = jnp.ones((NP, FEAT), f32)
    deg = _scatter_call(ones_feat, src_arr, dst_arr, zeros128)
    d1e, y, rst = _prep_call(deg[0], deg[1], x0, W1[0])

    layers = [(W1, b1), (W2, b2), (W3, b3), (W4, b4), (W5, b5)]
    for li, (W, b) in enumerate(layers):
        b2d = b.reshape(1, FEAT)
        s = _scatter_call(y, src_arr, dst_arr, zeros128)
        x1, y = _step1_call(s[0], s[1], d1e)
        s = _scatter_call(y, src_arr, dst_arr, zeros128)
        rst, = _mm_call(rst, x1, W[1])
        x2, y = _step23_call(s[0], s[1], d1e, x0)
        s = _scatter_call(y, src_arr, dst_arr, zeros128)
        rst, = _mm_call(rst, x2, W[2])
        x3, y = _step23_call(s[0], s[1], d1e, x1)
        s = _scatter_call(y, src_arr, dst_arr, zeros128)
        rst, = _mm_call(rst, x3, W[3])
        if li < 4:
            w0n = layers[li + 1][0][0]
            x0, y, rst = _layer_end_call(
                s[0], s[1], d1e, x2, rst, W[4], b2d, w0n)
        else:
            wl_pad = jnp.zeros((FEAT, FEAT), f32).at[:, :Wl.shape[1]].set(Wl)
            bl_pad = jnp.zeros((1, FEAT), f32).at[0, :bl.shape[0]].set(bl)
            out_p, = _final_end_call(
                s[0], s[1], d1e, x2, rst, W[4], b2d, wl_pad, bl_pad)

    return jnp.concatenate(
        [out_p[:HALF, :Wl.shape[1]],
         out_p[HALF + PADW:HALF + PADW + HALF, :Wl.shape[1]]], axis=0)
